# baseline (speedup 1.0000x reference)
import functools

import jax
import jax.numpy as jnp
from jax import lax
from jax.experimental import pallas as pl
from jax.experimental.pallas import tpu as pltpu
from jax.experimental.pallas import tpu_sc as plsc

_T_S = 16
_T_M = 8
_OUTER_THRE = 0.72
_LOGIT_THRE = 0.9444616088408514


def _sc_scatter_edges(edge_index, P, E):
    NS = 16
    epw = E // NS
    K = epw // 128
    SPLIT = 2048
    ZR = 4
    zwords = ZR * P

    mesh = plsc.VectorSubcoreMesh(core_axis_name="c", subcore_axis_name="s")

    @functools.partial(
        pl.kernel,
        mesh=mesh,
        out_type=jax.ShapeDtypeStruct(((P + 16) * P,), jnp.float32),
        scratch_types=[
            pltpu.VMEM((epw,), jnp.int32),
            pltpu.VMEM((epw,), jnp.int32),
            pltpu.VMEM((K, 128), jnp.int32),
            pltpu.VMEM((zwords,), jnp.float32),
            pltpu.VMEM((128,), jnp.float32),
            pltpu.SemaphoreType.DMA,
            pltpu.SemaphoreType.DMA,
            pltpu.SemaphoreType.DMA,
        ],
    )
    def sc_kernel(edges_hbm, s_out_hbm, e0_v, e1_v, idx_v, zbuf_v, ones_v,
                  zsem, sem, esem):
        c = lax.axis_index("c")
        s = lax.axis_index("s")

        base = s * epw
        ecp0 = pltpu.async_copy(edges_hbm.at[0, pl.ds(base, epw)], e0_v, esem)
        ecp1 = pltpu.async_copy(edges_hbm.at[1, pl.ds(base, epw)], e1_v, esem)

        def _zb(k, _):
            for u in range(4):
                zbuf_v[pl.ds(k * 64 + u * 16, 16)] = jnp.zeros((16,),
                                                               jnp.float32)
            return 0
        lax.fori_loop(0, zwords // 64, _zb, 0)

        main_base = jnp.where(c == 0, 0, SPLIT)
        rows_pw = jnp.where(c == 0, SPLIT // NS, (P - 64 - SPLIT) // NS)
        nzd = (P - 64 - SPLIT) // NS // ZR
        row0 = main_base + s * rows_pw
        zcopies = []
        for q in range(nzd):
            tail_row = (P - 64) + s * ZR
            r = jnp.where((c == 0) & (q == nzd - 1), tail_row, row0 + q * ZR)
            zcopies.append(
                pltpu.async_copy(zbuf_v, s_out_hbm.at[pl.ds(r * P, zwords)],
                                 zsem))
        ecp0.wait()
        ecp1.wait()
        land = P * P + c * (8 * P)
        for k in range(K):
            for l in range(8):
                o = k * 128 + l * 16
                a = e0_v[pl.ds(o, 16)]
                b = e1_v[pl.ds(o, 16)]
                r = b + _T_S
                flat = r * P + (a + _T_S)
                owned = jnp.where(c == 0, r < SPLIT, r >= SPLIT)
                idx_v[k, pl.ds(l * 16, 16)] = jnp.where(
                    owned, flat, land + (flat & 32767))
        for l in range(8):
            ones_v[pl.ds(l * 16, 16)] = jnp.full((16,), 1.0, jnp.float32)
        for cp in zcopies:
            cp.wait()
        plsc.subcore_barrier()
        copies = [
            pltpu.async_copy(ones_v, s_out_hbm.at[idx_v.at[k]], sem)
            for k in range(K)
        ]
        for cp in copies:
            cp.wait()

    return sc_kernel(edge_index)


def _tc_body(xp_ref, st_ref, w1_ref, w2_ref, sem_ref, cid_ref, out_ref,
             xn_s, y_s, z_s, du_s, *, T, P, normal, total, nj, ni):
    j = pl.program_id(0)
    i = pl.program_id(1)

    @pl.when((j == 0) & (i == 0))
    def _prologue():
        xp = xp_ref[...]
        nrm = jnp.sqrt(jnp.sum(xp * xp, axis=1, keepdims=True))
        xn_s[...] = xp / (nrm + 1e-12)
        y_s[...] = jnp.dot(xp, w1_ref[...], preferred_element_type=jnp.float32)
        z_s[...] = jnp.zeros_like(z_s)
        du_s[...] = jnp.zeros_like(du_s)
        out_ref[...] = jnp.zeros_like(out_ref)

    xj = xn_s[pl.ds(j * T, T), :]
    xi = xn_s[pl.ds(i * T, T), :]
    sim = lax.dot_general(xj, xi, (((1,), (1,)), ((), ())),
                          preferred_element_type=jnp.float32)
    v = jnp.where(st_ref[...] * 4.0 + sim > _LOGIT_THRE,
                  1.0, 0.0).astype(jnp.float32)
    yi = y_s[pl.ds(i * T, T), :]
    z_s[pl.ds(j * T, T), :] += jnp.dot(v, yi, preferred_element_type=jnp.float32)
    du_s[0:1, pl.ds(i * T, T)] += jnp.sum(v, axis=0, keepdims=True)

    @pl.when((j == nj - 1) & (i == ni - 1))
    def _epilogue():
        c = cid_ref[0, 0]
        w1 = w1_ref[...]
        ysem = jnp.dot(jnp.sum(sem_ref[...], axis=0, keepdims=True), w1,
                       preferred_element_type=jnp.float32)
        yc = jnp.dot(xp_ref[pl.ds(c, 1), :], w1,
                     preferred_element_type=jnp.float32)
        rows = lax.broadcasted_iota(jnp.int32, (P, 1), 0)
        maskn = rows < normal
        masks = (rows >= _T_S) & maskn
        z = z_s[...] + jnp.where(masks, ysem, 0.0)
        r = jnp.where(maskn, jnp.maximum(z, 0.0), 0.0)
        cols = lax.broadcasted_iota(jnp.int32, (1, P), 1)
        deg = du_s[...] + jnp.where(cols == c, float(_T_M), 0.0)
        h = jnp.dot(deg, r, preferred_element_type=jnp.float32)
        h = h + float(_T_M * (normal - _T_S)) * jnp.maximum(yc, 0.0)
        out_ref[...] = jnp.dot(h, w2_ref[...],
                               preferred_element_type=jnp.float32) / float(total)


def kernel(x, edge_index, center_id, structure_prompt, semantics_prompt, W1, W2):
    N, d = x.shape
    hid = W1.shape[1]
    E = edge_index.shape[1]
    normal = N + _T_S
    total = normal + _T_M
    T = 1408
    P = -(-normal // T) * T
    nj = ni = P // T

    new_x = jnp.concatenate([structure_prompt, x], axis=0)
    xp = jnp.zeros((P, d), jnp.float32).at[:normal, :].set(new_x)

    st_flat = _sc_scatter_edges(edge_index.astype(jnp.int32), P, E)
    st = st_flat.reshape(P + 16, P)

    cid = jnp.reshape(center_id + _T_S, (1, 1)).astype(jnp.int32)

    out = pl.pallas_call(
        functools.partial(_tc_body, T=T, P=P, normal=normal, total=total,
                          nj=nj, ni=ni),
        grid=(nj, ni),
        in_specs=[
            pl.BlockSpec((P, d), lambda j, i: (0, 0)),
            pl.BlockSpec((T, T), lambda j, i: (j, i)),
            pl.BlockSpec((d, hid), lambda j, i: (0, 0)),
            pl.BlockSpec((hid, hid), lambda j, i: (0, 0)),
            pl.BlockSpec((_T_M, d), lambda j, i: (0, 0)),
            pl.BlockSpec(memory_space=pltpu.SMEM),
        ],
        out_specs=pl.BlockSpec((1, hid), lambda j, i: (0, 0)),
        out_shape=jax.ShapeDtypeStruct((1, hid), jnp.float32),
        scratch_shapes=[
            pltpu.VMEM((P, d), jnp.float32),
            pltpu.VMEM((P, hid), jnp.float32),
            pltpu.VMEM((P, hid), jnp.float32),
            pltpu.VMEM((1, P), jnp.float32),
        ],
        compiler_params=pltpu.CompilerParams(
            vmem_limit_bytes=100 * 1024 * 1024),
    )(xp, st, W1, W2, semantics_prompt, cid)
    return out[0]


# speedup vs baseline: 1.3135x; 1.3135x over previous
import functools

import jax
import jax.numpy as jnp
from jax import lax
from jax.experimental import pallas as pl
from jax.experimental.pallas import tpu as pltpu
from jax.experimental.pallas import tpu_sc as plsc

_T_S = 16
_T_M = 8
_OUTER_THRE = 0.72
_LOGIT_THRE = 0.9444616088408514


def _sc_scatter_edges(edge_index, P, E):
    NS = 16
    epw = E // NS
    K = epw // 128
    ZR = 8
    zwords = ZR * P

    mesh = plsc.VectorSubcoreMesh(core_axis_name="c", subcore_axis_name="s",
                                  num_cores=1)

    @functools.partial(
        pl.kernel,
        mesh=mesh,
        out_type=jax.ShapeDtypeStruct(((P + 16) * P,), jnp.float32),
        scratch_types=[
            pltpu.VMEM((epw,), jnp.int32),
            pltpu.VMEM((epw,), jnp.int32),
            pltpu.VMEM((K, 128), jnp.int32),
            pltpu.VMEM((zwords,), jnp.float32),
            pltpu.VMEM((128,), jnp.float32),
            pltpu.SemaphoreType.DMA,
            pltpu.SemaphoreType.DMA,
            pltpu.SemaphoreType.DMA,
        ],
    )
    def sc_kernel(edges_hbm, s_out_hbm, e0_v, e1_v, idx_v, zbuf_v, ones_v,
                  zsem, sem, esem):
        s = lax.axis_index("s")

        base = s * epw
        ecp0 = pltpu.async_copy(edges_hbm.at[0, pl.ds(base, epw)], e0_v, esem)
        ecp1 = pltpu.async_copy(edges_hbm.at[1, pl.ds(base, epw)], e1_v, esem)

        def _zb(k, _):
            for u in range(4):
                zbuf_v[pl.ds(k * 64 + u * 16, 16)] = jnp.zeros((16,),
                                                               jnp.float32)
            return 0
        lax.fori_loop(0, zwords // 64, _zb, 0)

        rows_pw = P // NS
        nzd = rows_pw // ZR
        row0 = s * rows_pw
        zcopies = [
            pltpu.async_copy(zbuf_v,
                             s_out_hbm.at[pl.ds((row0 + q * ZR) * P, zwords)],
                             zsem)
            for q in range(nzd)
        ]
        ecp0.wait()
        ecp1.wait()
        for k in range(K):
            for l in range(8):
                o = k * 128 + l * 16
                a = e0_v[pl.ds(o, 16)]
                b = e1_v[pl.ds(o, 16)]
                idx_v[k, pl.ds(l * 16, 16)] = (b + _T_S) * P + (a + _T_S)
        for l in range(8):
            ones_v[pl.ds(l * 16, 16)] = jnp.full((16,), 1.0, jnp.float32)
        for cp in zcopies:
            cp.wait()
        plsc.subcore_barrier()
        copies = [
            pltpu.async_copy(ones_v, s_out_hbm.at[idx_v.at[k]], sem)
            for k in range(K)
        ]
        for cp in copies:
            cp.wait()

    return sc_kernel(edge_index)


def _tc_body(xp_ref, st_ref, w1_ref, w2_ref, sem_ref, cid_ref, out_ref,
             xn_s, y_s, z_s, du_s, *, T, P, normal, total, nj, ni):
    j = pl.program_id(0)
    i = pl.program_id(1)

    @pl.when((j == 0) & (i == 0))
    def _prologue():
        xp = xp_ref[...]
        nrm = jnp.sqrt(jnp.sum(xp * xp, axis=1, keepdims=True))
        xn_s[...] = xp / (nrm + 1e-12)
        y_s[...] = jnp.dot(xp, w1_ref[...], preferred_element_type=jnp.float32)
        z_s[...] = jnp.zeros_like(z_s)
        du_s[...] = jnp.zeros_like(du_s)
        out_ref[...] = jnp.zeros_like(out_ref)

    xj = xn_s[pl.ds(j * T, T), :]
    xi = xn_s[pl.ds(i * T, T), :]
    sim = lax.dot_general(xj, xi, (((1,), (1,)), ((), ())),
                          preferred_element_type=jnp.float32)
    v = jnp.where(st_ref[...] * 4.0 + sim > _LOGIT_THRE,
                  1.0, 0.0).astype(jnp.float32)
    yi = y_s[pl.ds(i * T, T), :]
    z_s[pl.ds(j * T, T), :] += jnp.dot(v, yi, preferred_element_type=jnp.float32)
    du_s[0:1, pl.ds(i * T, T)] += jnp.sum(v, axis=0, keepdims=True)

    @pl.when((j == nj - 1) & (i == ni - 1))
    def _epilogue():
        c = cid_ref[0, 0]
        w1 = w1_ref[...]
        ysem = jnp.dot(jnp.sum(sem_ref[...], axis=0, keepdims=True), w1,
                       preferred_element_type=jnp.float32)
        yc = jnp.dot(xp_ref[pl.ds(c, 1), :], w1,
                     preferred_element_type=jnp.float32)
        rows = lax.broadcasted_iota(jnp.int32, (P, 1), 0)
        maskn = rows < normal
        masks = (rows >= _T_S) & maskn
        z = z_s[...] + jnp.where(masks, ysem, 0.0)
        r = jnp.where(maskn, jnp.maximum(z, 0.0), 0.0)
        cols = lax.broadcasted_iota(jnp.int32, (1, P), 1)
        deg = du_s[...] + jnp.where(cols == c, float(_T_M), 0.0)
        h = jnp.dot(deg, r, preferred_element_type=jnp.float32)
        h = h + float(_T_M * (normal - _T_S)) * jnp.maximum(yc, 0.0)
        out_ref[...] = jnp.dot(h, w2_ref[...],
                               preferred_element_type=jnp.float32) / float(total)


def kernel(x, edge_index, center_id, structure_prompt, semantics_prompt, W1, W2):
    N, d = x.shape
    hid = W1.shape[1]
    E = edge_index.shape[1]
    normal = N + _T_S
    total = normal + _T_M
    T = 1408
    P = -(-normal // T) * T
    nj = ni = P // T

    new_x = jnp.concatenate([structure_prompt, x], axis=0)
    xp = jnp.zeros((P, d), jnp.float32).at[:normal, :].set(new_x)

    st_flat = _sc_scatter_edges(edge_index.astype(jnp.int32), P, E)
    st = st_flat.reshape(P + 16, P)

    cid = jnp.reshape(center_id + _T_S, (1, 1)).astype(jnp.int32)

    out = pl.pallas_call(
        functools.partial(_tc_body, T=T, P=P, normal=normal, total=total,
                          nj=nj, ni=ni),
        grid=(nj, ni),
        in_specs=[
            pl.BlockSpec((P, d), lambda j, i: (0, 0)),
            pl.BlockSpec((T, T), lambda j, i: (j, i)),
            pl.BlockSpec((d, hid), lambda j, i: (0, 0)),
            pl.BlockSpec((hid, hid), lambda j, i: (0, 0)),
            pl.BlockSpec((_T_M, d), lambda j, i: (0, 0)),
            pl.BlockSpec(memory_space=pltpu.SMEM),
        ],
        out_specs=pl.BlockSpec((1, hid), lambda j, i: (0, 0)),
        out_shape=jax.ShapeDtypeStruct((1, hid), jnp.float32),
        scratch_shapes=[
            pltpu.VMEM((P, d), jnp.float32),
            pltpu.VMEM((P, hid), jnp.float32),
            pltpu.VMEM((P, hid), jnp.float32),
            pltpu.VMEM((1, P), jnp.float32),
        ],
        compiler_params=pltpu.CompilerParams(
            vmem_limit_bytes=100 * 1024 * 1024),
    )(xp, st, W1, W2, semantics_prompt, cid)
    return out[0]
